# Initial kernel scaffold; baseline (speedup 1.0000x reference)
import jax
import jax.numpy as jnp
from jax.experimental import pallas as pl


def kernel(xyz):
    raise NotImplementedError("write your pallas kernel here")


# speedup vs baseline: 1.5731x; 1.5731x over previous
import jax
import jax.numpy as jnp
from jax import lax
from jax.experimental import pallas as pl

NG = 128
KS = 32
CP = 8


def _fps_body(xyz_ref, cen_ref):
    pts = xyz_ref[0]
    n = pts.shape[1]
    lane = lax.broadcasted_iota(jnp.int32, (1, n), 1)
    lane_g = lax.broadcasted_iota(jnp.int32, (CP, NG), 1)

    def step(i, carry):
        idxf, dists, acc = carry
        onehot = (lane == idxf).astype(jnp.float32)
        c = jnp.sum(pts * onehot, axis=1, keepdims=True)
        acc = jnp.where(lane_g == i, c, acc)
        diff = pts - c
        sq = diff * diff
        d = (sq[0:1] + sq[1:2]) + sq[2:3]
        dists = jnp.minimum(dists, d)
        m = jnp.max(dists, axis=1, keepdims=True)
        idxf = jnp.min(jnp.where(dists == m, lane, n),
                       axis=1, keepdims=True)
        return idxf, dists, acc

    idxf0 = jnp.zeros((1, 1), jnp.int32)
    dists0 = jnp.full((1, pts.shape[1]), 1e10, jnp.float32)
    acc0 = jnp.zeros((CP, NG), jnp.float32)
    _, _, acc = lax.fori_loop(0, NG, step, (idxf0, dists0, acc0))
    cen_ref[0] = acc


def _knn_body(pts_ref, cen_ref, org_ref, ctr_ref):
    pts = pts_ref[0, 0]
    cen = cen_ref[0]
    n = pts.shape[1]

    sq = pts * pts
    p2 = (sq[0:1] + sq[1:2]) + sq[2:3]

    ii = lax.broadcasted_iota(jnp.int32, (NG, NG), 0)
    jj = lax.broadcasted_iota(jnp.int32, (NG, NG), 1)
    eye = (ii == jj).astype(jnp.float32)
    ct = lax.dot_general(eye, cen, (((1,), (1,)), ((), ())),
                         preferred_element_type=jnp.float32,
                         precision=lax.Precision.HIGHEST)
    cs = ct * ct
    c2 = (cs[:, 0:1] + cs[:, 1:2]) + cs[:, 2:3]

    e = lax.dot_general(cen, pts, (((0,), (0,)), ((), ())),
                        preferred_element_type=jnp.float32)
    d2 = (c2 + p2) - 2.0 * e

    lane = lax.broadcasted_iota(jnp.int32, (NG, n), 1)
    slot = lax.broadcasted_iota(jnp.int32, (NG, KS * CP), 1) // CP

    def step(k, carry):
        d2, org_acc, ctr_acc = carry
        mn = jnp.min(d2, axis=1, keepdims=True)
        idxf = jnp.min(jnp.where(d2 == mn, lane, n),
                       axis=1, keepdims=True)
        sel = lane == idxf
        d2 = jnp.where(sel, jnp.inf, d2)
        p = lax.dot_general(sel.astype(jnp.float32), pts,
                            (((1,), (1,)), ((), ())),
                            preferred_element_type=jnp.float32,
                            precision=lax.Precision.HIGHEST)
        p_t = jnp.concatenate([p] * KS, axis=1)
        c_t = jnp.concatenate([p - ct] * KS, axis=1)
        org_acc = jnp.where(slot == k, p_t, org_acc)
        ctr_acc = jnp.where(slot == k, c_t, ctr_acc)
        return d2, org_acc, ctr_acc

    acc0 = jnp.zeros((NG, KS * CP), jnp.float32)
    _, org_acc, ctr_acc = lax.fori_loop(0, KS, step, (d2, acc0, acc0))
    org_ref[0] = org_acc
    ctr_ref[0] = ctr_acc


def kernel(xyz):
    V, B, N, _ = xyz.shape
    xyz_t = jnp.transpose(xyz, (0, 1, 3, 2))
    xyz_t = jnp.concatenate(
        [xyz_t, jnp.zeros((V, B, CP - 3, N), xyz.dtype)], axis=2)

    centers = pl.pallas_call(
        _fps_body,
        grid=(B,),
        in_specs=[pl.BlockSpec((1, CP, N), lambda b: (b, 0, 0))],
        out_specs=pl.BlockSpec((1, CP, NG), lambda b: (b, 0, 0)),
        out_shape=jax.ShapeDtypeStruct((B, CP, NG), jnp.float32),
    )(xyz_t[0])

    org, ctr = pl.pallas_call(
        _knn_body,
        grid=(B * V,),
        in_specs=[
            pl.BlockSpec((1, 1, CP, N), lambda p: (p % V, p // V, 0, 0)),
            pl.BlockSpec((1, CP, NG), lambda p: (p // V, 0, 0)),
        ],
        out_specs=[
            pl.BlockSpec((1, NG, KS * CP), lambda p: (p, 0, 0)),
            pl.BlockSpec((1, NG, KS * CP), lambda p: (p, 0, 0)),
        ],
        out_shape=[
            jax.ShapeDtypeStruct((B * V, NG, KS * CP), jnp.float32),
            jax.ShapeDtypeStruct((B * V, NG, KS * CP), jnp.float32),
        ],
    )(xyz_t, centers)

    neighborhood_org = org.reshape(B * V, NG, KS, CP)[..., :3]
    neighborhood = ctr.reshape(B * V, NG, KS, CP)[..., :3]
    cen3 = jnp.transpose(centers, (0, 2, 1))[..., :3]
    center_flat = jnp.broadcast_to(
        cen3[:, None], (B, V, NG, 3)).reshape(B * V, NG, 3)
    return neighborhood, center_flat, neighborhood_org


# speedup vs baseline: 1.7763x; 1.1292x over previous
import jax
import jax.numpy as jnp
from jax import lax
from jax.experimental import pallas as pl

NG = 128
KS = 32
CP = 8


def _fps_body(x_ref, y_ref, z_ref, cx_ref, cy_ref, cz_ref):
    x, y, z = x_ref[...], y_ref[...], z_ref[...]
    b, n = x.shape
    lane = lax.broadcasted_iota(jnp.int32, (1, n), 1)
    lane_g = lax.broadcasted_iota(jnp.int32, (b, NG), 1)

    def step(i, carry):
        idxf, dists, ax, ay, az = carry
        onehot = jnp.where(lane == idxf, 1.0, 0.0)
        cx = jnp.sum(x * onehot, axis=1, keepdims=True)
        cy = jnp.sum(y * onehot, axis=1, keepdims=True)
        cz = jnp.sum(z * onehot, axis=1, keepdims=True)
        ax = jnp.where(lane_g == i, cx, ax)
        ay = jnp.where(lane_g == i, cy, ay)
        az = jnp.where(lane_g == i, cz, az)
        dx = x - cx
        dy = y - cy
        dz = z - cz
        d = (dx * dx + dy * dy) + dz * dz
        dists = jnp.minimum(dists, d)
        m = jnp.max(dists, axis=1, keepdims=True)
        idxf = jnp.min(jnp.where(dists == m, lane, n),
                       axis=1, keepdims=True)
        return idxf, dists, ax, ay, az

    idxf0 = jnp.zeros((b, 1), jnp.int32)
    dists0 = jnp.full((b, n), 1e10, jnp.float32)
    acc0 = jnp.zeros((b, NG), jnp.float32)
    _, _, ax, ay, az = lax.fori_loop(
        0, NG, step, (idxf0, dists0, acc0, acc0, acc0))
    cx_ref[...] = ax
    cy_ref[...] = ay
    cz_ref[...] = az


def _knn_body(pts_ref, cen_ref, org_ref, ctr_ref):
    pts = pts_ref[0, 0]
    cen = cen_ref[0]
    n = pts.shape[1]

    sq = pts * pts
    p2 = (sq[0:1] + sq[1:2]) + sq[2:3]

    ii = lax.broadcasted_iota(jnp.int32, (NG, NG), 0)
    jj = lax.broadcasted_iota(jnp.int32, (NG, NG), 1)
    eye = (ii == jj).astype(jnp.float32)
    ct = lax.dot_general(eye, cen, (((1,), (1,)), ((), ())),
                         preferred_element_type=jnp.float32,
                         precision=lax.Precision.HIGHEST)
    cs = ct * ct
    c2 = (cs[:, 0:1] + cs[:, 1:2]) + cs[:, 2:3]

    e = lax.dot_general(cen, pts, (((0,), (0,)), ((), ())),
                        preferred_element_type=jnp.float32)
    d2 = (c2 + p2) - 2.0 * e

    lane = lax.broadcasted_iota(jnp.int32, (NG, n), 1)
    slot = lax.broadcasted_iota(jnp.int32, (NG, KS * CP), 1) // CP

    def step(k, carry):
        d2, org_acc, ctr_acc = carry
        mn = jnp.min(d2, axis=1, keepdims=True)
        idxf = jnp.min(jnp.where(d2 == mn, lane, n),
                       axis=1, keepdims=True)
        self = jnp.where(lane == idxf, 1.0, 0.0)
        d2 = d2 + self * 3e38
        p = lax.dot_general(self, pts,
                            (((1,), (1,)), ((), ())),
                            preferred_element_type=jnp.float32,
                            precision=lax.Precision.HIGHEST)
        p_t = jnp.concatenate([p] * KS, axis=1)
        c_t = jnp.concatenate([p - ct] * KS, axis=1)
        org_acc = jnp.where(slot == k, p_t, org_acc)
        ctr_acc = jnp.where(slot == k, c_t, ctr_acc)
        return d2, org_acc, ctr_acc

    acc0 = jnp.zeros((NG, KS * CP), jnp.float32)
    _, org_acc, ctr_acc = lax.fori_loop(0, KS, step, (d2, acc0, acc0))
    org_ref[0] = org_acc
    ctr_ref[0] = ctr_acc


def kernel(xyz):
    V, B, N, _ = xyz.shape
    xyz_t = jnp.transpose(xyz, (0, 1, 3, 2))
    xyz_t = jnp.concatenate(
        [xyz_t, jnp.zeros((V, B, CP - 3, N), xyz.dtype)], axis=2)

    cx, cy, cz = pl.pallas_call(
        _fps_body,
        out_shape=[jax.ShapeDtypeStruct((B, NG), jnp.float32)] * 3,
    )(xyz_t[0, :, 0], xyz_t[0, :, 1], xyz_t[0, :, 2])
    centers = jnp.stack(
        [cx, cy, cz, jnp.zeros_like(cx), jnp.zeros_like(cx),
         jnp.zeros_like(cx), jnp.zeros_like(cx), jnp.zeros_like(cx)],
        axis=1)

    org, ctr = pl.pallas_call(
        _knn_body,
        grid=(B * V,),
        in_specs=[
            pl.BlockSpec((1, 1, CP, N), lambda p: (p % V, p // V, 0, 0)),
            pl.BlockSpec((1, CP, NG), lambda p: (p // V, 0, 0)),
        ],
        out_specs=[
            pl.BlockSpec((1, NG, KS * CP), lambda p: (p, 0, 0)),
            pl.BlockSpec((1, NG, KS * CP), lambda p: (p, 0, 0)),
        ],
        out_shape=[
            jax.ShapeDtypeStruct((B * V, NG, KS * CP), jnp.float32),
            jax.ShapeDtypeStruct((B * V, NG, KS * CP), jnp.float32),
        ],
    )(xyz_t, centers)

    neighborhood_org = org.reshape(B * V, NG, KS, CP)[..., :3]
    neighborhood = ctr.reshape(B * V, NG, KS, CP)[..., :3]
    cen3 = jnp.transpose(centers, (0, 2, 1))[..., :3]
    center_flat = jnp.broadcast_to(
        cen3[:, None], (B, V, NG, 3)).reshape(B * V, NG, 3)
    return neighborhood, center_flat, neighborhood_org
